# baseline (speedup 1.0000x reference)
import functools

import jax
import jax.numpy as jnp
from jax.experimental import pallas as pl

_TOP_K = 64


def _topk_mask_kernel(x_ref, o_ref, *, k):
    x = x_ref[...]
    u = jax.lax.bitcast_convert_type(x, jnp.uint32)
    top = jnp.uint32(0x80000000)
    key = jnp.where(u >= top, ~u, u | top)

    rows, d = x.shape
    ones8 = jnp.ones((d, 8), jnp.float32)
    kf = jnp.float32(k)
    lo = jnp.zeros((rows, 1), jnp.uint32)
    for b in range(31, -1, -1):
        cand = lo | jnp.uint32(1 << b)
        ind = (key >= cand).astype(jnp.float32)
        cnt = jax.lax.dot_general(
            ind, ones8, (((1,), (0,)), ((), ())),
            preferred_element_type=jnp.float32,
        )[:, 0:1]
        lo = jnp.where(cnt >= kf, cand, lo)

    ut = jnp.where(lo >= top, lo ^ top, ~lo)
    t = jax.lax.bitcast_convert_type(ut, jnp.float32)

    gt = x > t
    eq = x == t
    n_gt = jnp.sum(gt.astype(jnp.int32), axis=1, keepdims=True)
    n_eq = jnp.sum(eq.astype(jnp.int32), axis=1, keepdims=True)
    exact = jnp.sum(((n_gt + n_eq) > k).astype(jnp.int32)) == 0

    @pl.when(exact)
    def _():
        o_ref[...] = jnp.where(x >= t, x, 0.0)

    @pl.when(jnp.logical_not(exact))
    def _():
        n_keep = k - n_gt
        idx = jax.lax.broadcasted_iota(jnp.int32, x.shape, 1)
        cut = jnp.zeros((rows, 1), jnp.int32)
        for b in range(16, -1, -1):
            cand = cut | jnp.int32(1 << b)
            cnt_lt = jnp.sum(
                (eq & (idx < cand)).astype(jnp.int32), axis=1, keepdims=True
            )
            cut = jnp.where(cnt_lt <= n_keep, cand, cut)
        keep = gt | (eq & (idx < cut))
        o_ref[...] = jnp.where(keep, x, 0.0)


def kernel(x):
    bsz, d_sae = x.shape
    k = min(_TOP_K, d_sae)
    rows_per_block = 8
    grid = bsz // rows_per_block
    return pl.pallas_call(
        functools.partial(_topk_mask_kernel, k=k),
        out_shape=jax.ShapeDtypeStruct((bsz, d_sae), x.dtype),
        grid=(grid,),
        in_specs=[pl.BlockSpec((rows_per_block, d_sae), lambda i: (i, 0))],
        out_specs=pl.BlockSpec((rows_per_block, d_sae), lambda i: (i, 0)),
    )(x)


# speedup vs baseline: 6.3286x; 6.3286x over previous
import functools

import jax
import jax.numpy as jnp
from jax.experimental import pallas as pl

_TOP_K = 64


def _topk_mask_kernel(x_ref, o_ref, *, k):
    x = x_ref[...]
    u = jax.lax.bitcast_convert_type(x, jnp.uint32)
    top = jnp.uint32(0x80000000)
    key = jnp.where(u >= top, ~u, u | top)

    rows, d = x.shape
    one16 = jnp.int16(1)
    zero16 = jnp.int16(0)

    def count16(ind16):
        c = ind16
        w = d
        while w > 256:
            w //= 2
            c = c[:, :w] + c[:, w : 2 * w]
        return jnp.sum(c.astype(jnp.int32), axis=1, keepdims=True)

    hi16 = ((key >> jnp.uint32(16)).astype(jnp.int32) - 32768).astype(jnp.int16)
    low16 = ((key & jnp.uint32(0xFFFF)).astype(jnp.int32) - 32768).astype(
        jnp.int16
    )

    lo1 = jnp.zeros((rows, 1), jnp.int32)
    for b in range(15, -1, -1):
        cand = lo1 | jnp.int32(1 << b)
        c16 = (cand - 32768).astype(jnp.int16)
        ind = jnp.where(hi16 >= c16, one16, zero16)
        lo1 = jnp.where(count16(ind) >= k, cand, lo1)
    t16 = (lo1 - 32768).astype(jnp.int16)

    n_above = count16(jnp.where(hi16 > t16, one16, zero16))
    mlow = jnp.where(hi16 == t16, low16, jnp.int16(-32768))
    lo2 = jnp.zeros((rows, 1), jnp.int32)
    for b in range(15, -1, -1):
        cand = lo2 | jnp.int32(1 << b)
        c16 = (cand - 32768).astype(jnp.int16)
        ind = jnp.where(mlow >= c16, one16, zero16)
        lo2 = jnp.where(n_above + count16(ind) >= k, cand, lo2)

    tkey = (lo1.astype(jnp.uint32) << jnp.uint32(16)) | lo2.astype(jnp.uint32)
    ut = jnp.where(tkey >= top, tkey ^ top, ~tkey)
    t = jax.lax.bitcast_convert_type(ut, jnp.float32)

    gt = x > t
    eq = x == t
    n_gt = jnp.sum(gt.astype(jnp.int32), axis=1, keepdims=True)
    n_eq = jnp.sum(eq.astype(jnp.int32), axis=1, keepdims=True)
    exact = jnp.sum(((n_gt + n_eq) > k).astype(jnp.int32)) == 0

    @pl.when(exact)
    def _():
        o_ref[...] = jnp.where(x >= t, x, 0.0)

    @pl.when(jnp.logical_not(exact))
    def _():
        n_keep = k - n_gt
        idx = jax.lax.broadcasted_iota(jnp.int32, x.shape, 1)
        cut = jnp.zeros((rows, 1), jnp.int32)
        for b in range(16, -1, -1):
            cand = cut | jnp.int32(1 << b)
            cnt_lt = jnp.sum(
                (eq & (idx < cand)).astype(jnp.int32), axis=1, keepdims=True
            )
            cut = jnp.where(cnt_lt <= n_keep, cand, cut)
        keep = gt | (eq & (idx < cut))
        o_ref[...] = jnp.where(keep, x, 0.0)


def kernel(x):
    bsz, d_sae = x.shape
    k = min(_TOP_K, d_sae)
    rows_per_block = 8
    grid = bsz // rows_per_block
    return pl.pallas_call(
        functools.partial(_topk_mask_kernel, k=k),
        out_shape=jax.ShapeDtypeStruct((bsz, d_sae), x.dtype),
        grid=(grid,),
        in_specs=[pl.BlockSpec((rows_per_block, d_sae), lambda i: (i, 0))],
        out_specs=pl.BlockSpec((rows_per_block, d_sae), lambda i: (i, 0)),
    )(x)


# speedup vs baseline: 6.3435x; 1.0023x over previous
import functools

import jax
import jax.numpy as jnp
from jax.experimental import pallas as pl

_TOP_K = 64


def _topk_mask_kernel(x_ref, o_ref, *, k):
    x = x_ref[...]
    i32 = jax.lax.bitcast_convert_type(x, jnp.int32)
    ki = i32 ^ ((i32 >> jnp.int32(31)) & jnp.int32(0x7FFFFFFF))

    rows, d = x.shape
    one16 = jnp.int16(1)
    zero16 = jnp.int16(0)

    def count16(ind16):
        c = ind16
        w = d
        while w > 256:
            w //= 2
            c = c[:, :w] + c[:, w : 2 * w]
        return jnp.sum(c.astype(jnp.int32), axis=1, keepdims=True)

    hi16 = (ki >> jnp.int32(16)).astype(jnp.int16)
    low16 = (ki ^ jnp.int32(0x8000)).astype(jnp.int16)

    lo1 = jnp.zeros((rows, 1), jnp.int32)
    for b in range(15, -1, -1):
        cand = lo1 | jnp.int32(1 << b)
        c16 = (cand - 32768).astype(jnp.int16)
        ind = jnp.where(hi16 >= c16, one16, zero16)
        lo1 = jnp.where(count16(ind) >= k, cand, lo1)
    t16 = (lo1 - 32768).astype(jnp.int16)

    n_above = count16(jnp.where(hi16 > t16, one16, zero16))
    mlow = jnp.where(hi16 == t16, low16, jnp.int16(-32768))
    lo2 = jnp.zeros((rows, 1), jnp.int32)
    for b in range(15, -1, -1):
        cand = lo2 | jnp.int32(1 << b)
        c16 = (cand - 32768).astype(jnp.int16)
        ind = jnp.where(mlow >= c16, one16, zero16)
        lo2 = jnp.where(n_above + count16(ind) >= k, cand, lo2)

    tki = ((lo1 - 32768) << jnp.int32(16)) | lo2
    ti = tki ^ ((tki >> jnp.int32(31)) & jnp.int32(0x7FFFFFFF))
    t = jax.lax.bitcast_convert_type(ti, jnp.float32)

    gt = x > t
    eq = x == t
    n_gt = jnp.sum(gt.astype(jnp.int32), axis=1, keepdims=True)
    n_eq = jnp.sum(eq.astype(jnp.int32), axis=1, keepdims=True)
    exact = jnp.sum(((n_gt + n_eq) > k).astype(jnp.int32)) == 0

    @pl.when(exact)
    def _():
        o_ref[...] = jnp.where(x >= t, x, 0.0)

    @pl.when(jnp.logical_not(exact))
    def _():
        n_keep = k - n_gt
        idx = jax.lax.broadcasted_iota(jnp.int32, x.shape, 1)
        cut = jnp.zeros((rows, 1), jnp.int32)
        for b in range(16, -1, -1):
            cand = cut | jnp.int32(1 << b)
            cnt_lt = jnp.sum(
                (eq & (idx < cand)).astype(jnp.int32), axis=1, keepdims=True
            )
            cut = jnp.where(cnt_lt <= n_keep, cand, cut)
        keep = gt | (eq & (idx < cut))
        o_ref[...] = jnp.where(keep, x, 0.0)


def kernel(x):
    bsz, d_sae = x.shape
    k = min(_TOP_K, d_sae)
    rows_per_block = 8
    grid = bsz // rows_per_block
    return pl.pallas_call(
        functools.partial(_topk_mask_kernel, k=k),
        out_shape=jax.ShapeDtypeStruct((bsz, d_sae), x.dtype),
        grid=(grid,),
        in_specs=[pl.BlockSpec((rows_per_block, d_sae), lambda i: (i, 0))],
        out_specs=pl.BlockSpec((rows_per_block, d_sae), lambda i: (i, 0)),
    )(x)


# speedup vs baseline: 6.5172x; 1.0274x over previous
import functools

import jax
import jax.numpy as jnp
from jax.experimental import pallas as pl

_TOP_K = 64


def _topk_mask_kernel(x_ref, o_ref, *, k):
    x = x_ref[...]
    i32 = jax.lax.bitcast_convert_type(x, jnp.int32)
    ki = i32 ^ ((i32 >> jnp.int32(31)) & jnp.int32(0x7FFFFFFF))

    rows, d = x.shape
    one16 = jnp.int16(1)
    zero16 = jnp.int16(0)

    def count16(ind16):
        c = ind16
        w = d
        while w > 256:
            w //= 2
            c = c[:, :w] + c[:, w : 2 * w]
        return jnp.sum(c.astype(jnp.int32), axis=1, keepdims=True)

    hi16 = (ki >> jnp.int32(16)).astype(jnp.int16)
    low16 = (ki ^ jnp.int32(0x8000)).astype(jnp.int16)

    lo1 = jnp.zeros((rows, 1), jnp.int32)
    for b in range(15, -1, -1):
        cand = lo1 | jnp.int32(1 << b)
        c16 = (cand - 32768).astype(jnp.int16)
        ind = jnp.where(hi16 >= c16, one16, zero16)
        lo1 = jnp.where(count16(ind) >= k, cand, lo1)
    t16 = (lo1 - 32768).astype(jnp.int16)

    n_above = count16(jnp.where(hi16 > t16, one16, zero16))
    mlow = jnp.where(hi16 == t16, low16, jnp.int16(-32768))
    lo2 = jnp.zeros((rows, 1), jnp.int32)
    for b in range(15, -1, -1):
        cand = lo2 | jnp.int32(1 << b)
        c16 = (cand - 32768).astype(jnp.int16)
        ind = jnp.where(mlow >= c16, one16, zero16)
        lo2 = jnp.where(n_above + count16(ind) >= k, cand, lo2)

    tki = ((lo1 - 32768) << jnp.int32(16)) | lo2
    ti = tki ^ ((tki >> jnp.int32(31)) & jnp.int32(0x7FFFFFFF))
    t = jax.lax.bitcast_convert_type(ti, jnp.float32)

    tlow16 = (lo2 - 32768).astype(jnp.int16)
    n_gt = n_above + count16(jnp.where(mlow > tlow16, one16, zero16))
    n_ge = n_above + count16(jnp.where(mlow >= tlow16, one16, zero16))
    low_zero = jnp.sum((lo2 == 0).astype(jnp.int32)) > 0
    exact = (jnp.sum((n_ge > k).astype(jnp.int32)) == 0) & jnp.logical_not(
        low_zero
    )

    @pl.when(exact)
    def _():
        o_ref[...] = jnp.where(x >= t, x, 0.0)

    @pl.when(jnp.logical_not(exact))
    def _():
        gt = x > t
        eq = x == t
        n_gt_x = jnp.sum(gt.astype(jnp.int32), axis=1, keepdims=True)
        n_keep = k - n_gt_x
        idx = jax.lax.broadcasted_iota(jnp.int32, x.shape, 1)
        cut = jnp.zeros((rows, 1), jnp.int32)
        for b in range(16, -1, -1):
            cand = cut | jnp.int32(1 << b)
            cnt_lt = jnp.sum(
                (eq & (idx < cand)).astype(jnp.int32), axis=1, keepdims=True
            )
            cut = jnp.where(cnt_lt <= n_keep, cand, cut)
        keep = gt | (eq & (idx < cut))
        o_ref[...] = jnp.where(keep, x, 0.0)


def kernel(x):
    bsz, d_sae = x.shape
    k = min(_TOP_K, d_sae)
    rows_per_block = 8
    grid = bsz // rows_per_block
    return pl.pallas_call(
        functools.partial(_topk_mask_kernel, k=k),
        out_shape=jax.ShapeDtypeStruct((bsz, d_sae), x.dtype),
        grid=(grid,),
        in_specs=[pl.BlockSpec((rows_per_block, d_sae), lambda i: (i, 0))],
        out_specs=pl.BlockSpec((rows_per_block, d_sae), lambda i: (i, 0)),
    )(x)


# speedup vs baseline: 12.0498x; 1.8489x over previous
import functools

import jax
import jax.numpy as jnp
from jax.experimental import pallas as pl

_TOP_K = 64


def _topk_mask_kernel(x_ref, o_ref, *, k):
    x = x_ref[...]
    i32 = jax.lax.bitcast_convert_type(x, jnp.int32)
    ki = i32 ^ ((i32 >> jnp.int32(31)) & jnp.int32(0x7FFFFFFF))

    rows, d = x.shape
    one16 = jnp.int16(1)
    zero16 = jnp.int16(0)

    def count16(ind16):
        c = ind16
        w = d
        while w > 256:
            w //= 2
            c = c[:, :w] + c[:, w : 2 * w]
        return jnp.sum(c.astype(jnp.int32), axis=1, keepdims=True)

    hi16 = (ki >> jnp.int32(16)).astype(jnp.int16)
    low16 = (ki ^ jnp.int32(0x8000)).astype(jnp.int16)

    lo1 = jnp.zeros((rows, 1), jnp.int32)
    for b in range(15, -1, -1):
        cand = lo1 | jnp.int32(1 << b)
        c16 = (cand - 32768).astype(jnp.int16)
        ind = jnp.where(hi16 >= c16, one16, zero16)
        lo1 = jnp.where(count16(ind) >= k, cand, lo1)
    t16 = (lo1 - 32768).astype(jnp.int16)

    n_above = count16(jnp.where(hi16 > t16, one16, zero16))
    mlow = jnp.where(hi16 == t16, low16, jnp.int16(-32768))
    lo2 = jnp.zeros((rows, 1), jnp.int32)
    for b in range(15, -1, -1):
        cand = lo2 | jnp.int32(1 << b)
        c16 = (cand - 32768).astype(jnp.int16)
        ind = jnp.where(mlow >= c16, one16, zero16)
        lo2 = jnp.where(n_above + count16(ind) >= k, cand, lo2)

    tki = ((lo1 - 32768) << jnp.int32(16)) | lo2
    ti = tki ^ ((tki >> jnp.int32(31)) & jnp.int32(0x7FFFFFFF))
    t = jax.lax.bitcast_convert_type(ti, jnp.float32)

    tlow16 = (lo2 - 32768).astype(jnp.int16)
    n_gt = n_above + count16(jnp.where(mlow > tlow16, one16, zero16))
    n_ge = n_above + count16(jnp.where(mlow >= tlow16, one16, zero16))
    low_zero = jnp.sum((lo2 == 0).astype(jnp.int32)) > 0
    exact = (jnp.sum((n_ge > k).astype(jnp.int32)) == 0) & jnp.logical_not(
        low_zero
    )

    @pl.when(exact)
    def _():
        o_ref[...] = jnp.where(x >= t, x, 0.0)

    @pl.when(jnp.logical_not(exact))
    def _():
        gt = x > t
        eq = x == t
        n_gt_x = jnp.sum(gt.astype(jnp.int32), axis=1, keepdims=True)
        n_keep = k - n_gt_x
        idx = jax.lax.broadcasted_iota(jnp.int32, x.shape, 1)
        cut = jnp.zeros((rows, 1), jnp.int32)
        for b in range(16, -1, -1):
            cand = cut | jnp.int32(1 << b)
            cnt_lt = jnp.sum(
                (eq & (idx < cand)).astype(jnp.int32), axis=1, keepdims=True
            )
            cut = jnp.where(cnt_lt <= n_keep, cand, cut)
        keep = gt | (eq & (idx < cut))
        o_ref[...] = jnp.where(keep, x, 0.0)


def kernel(x):
    bsz, d_sae = x.shape
    k = min(_TOP_K, d_sae)
    rows_per_block = 16
    grid = bsz // rows_per_block
    return pl.pallas_call(
        functools.partial(_topk_mask_kernel, k=k),
        out_shape=jax.ShapeDtypeStruct((bsz, d_sae), x.dtype),
        grid=(grid,),
        in_specs=[pl.BlockSpec((rows_per_block, d_sae), lambda i: (i, 0))],
        out_specs=pl.BlockSpec((rows_per_block, d_sae), lambda i: (i, 0)),
    )(x)


# speedup vs baseline: 14.1302x; 1.1727x over previous
import functools

import jax
import jax.numpy as jnp
from jax.experimental import pallas as pl

_TOP_K = 64


def _topk_mask_kernel(x_ref, o_ref, *, k):
    x = x_ref[...]
    i32 = jax.lax.bitcast_convert_type(x, jnp.int32)
    ki = i32 ^ ((i32 >> jnp.int32(31)) & jnp.int32(0x7FFFFFFF))

    rows, d = x.shape
    one16 = jnp.int16(1)
    zero16 = jnp.int16(0)

    def count16(ind16):
        c = ind16
        w = d
        while w > 256:
            w //= 2
            c = c[:, :w] + c[:, w : 2 * w]
        return jnp.sum(c.astype(jnp.int32), axis=1, keepdims=True)

    hi16 = (ki >> jnp.int32(16)).astype(jnp.int16)
    low16 = (ki ^ jnp.int32(0x8000)).astype(jnp.int16)

    lo1 = jnp.zeros((rows, 1), jnp.int32)
    for b in range(15, -1, -1):
        cand = lo1 | jnp.int32(1 << b)
        c16 = (cand - 32768).astype(jnp.int16)
        ind = jnp.where(hi16 >= c16, one16, zero16)
        lo1 = jnp.where(count16(ind) >= k, cand, lo1)
    t16 = (lo1 - 32768).astype(jnp.int16)

    n_above = count16(jnp.where(hi16 > t16, one16, zero16))
    mlow = jnp.where(hi16 == t16, low16, jnp.int16(-32768))
    lo2 = jnp.zeros((rows, 1), jnp.int32)
    for b in range(15, -1, -1):
        cand = lo2 | jnp.int32(1 << b)
        c16 = (cand - 32768).astype(jnp.int16)
        ind = jnp.where(mlow >= c16, one16, zero16)
        lo2 = jnp.where(n_above + count16(ind) >= k, cand, lo2)

    tki = ((lo1 - 32768) << jnp.int32(16)) | lo2
    ti = tki ^ ((tki >> jnp.int32(31)) & jnp.int32(0x7FFFFFFF))
    t = jax.lax.bitcast_convert_type(ti, jnp.float32)

    tlow16 = (lo2 - 32768).astype(jnp.int16)
    n_gt = n_above + count16(jnp.where(mlow > tlow16, one16, zero16))
    n_ge = n_above + count16(jnp.where(mlow >= tlow16, one16, zero16))
    low_zero = jnp.sum((lo2 == 0).astype(jnp.int32)) > 0
    exact = (jnp.sum((n_ge > k).astype(jnp.int32)) == 0) & jnp.logical_not(
        low_zero
    )

    @pl.when(exact)
    def _():
        o_ref[...] = jnp.where(x >= t, x, 0.0)

    @pl.when(jnp.logical_not(exact))
    def _():
        gt = x > t
        eq = x == t
        n_gt_x = jnp.sum(gt.astype(jnp.int32), axis=1, keepdims=True)
        n_keep = k - n_gt_x
        idx = jax.lax.broadcasted_iota(jnp.int32, x.shape, 1)
        cut = jnp.zeros((rows, 1), jnp.int32)
        for b in range(16, -1, -1):
            cand = cut | jnp.int32(1 << b)
            cnt_lt = jnp.sum(
                (eq & (idx < cand)).astype(jnp.int32), axis=1, keepdims=True
            )
            cut = jnp.where(cnt_lt <= n_keep, cand, cut)
        keep = gt | (eq & (idx < cut))
        o_ref[...] = jnp.where(keep, x, 0.0)


def kernel(x):
    bsz, d_sae = x.shape
    k = min(_TOP_K, d_sae)
    rows_per_block = 32
    grid = bsz // rows_per_block
    return pl.pallas_call(
        functools.partial(_topk_mask_kernel, k=k),
        out_shape=jax.ShapeDtypeStruct((bsz, d_sae), x.dtype),
        grid=(grid,),
        in_specs=[pl.BlockSpec((rows_per_block, d_sae), lambda i: (i, 0))],
        out_specs=pl.BlockSpec((rows_per_block, d_sae), lambda i: (i, 0)),
    )(x)
